# Initial kernel scaffold; baseline (speedup 1.0000x reference)
import jax
import jax.numpy as jnp
from jax.experimental import pallas as pl


def kernel(sp_fea, sp_xyz, o_p_fea, p_xyz, c2p_idx_abs, c2p_idx, cluster_idx, offset, sp_offset, fea_w1, fea_b1, fea_g1, fea_be1, fea_w2, fea_b2, xyz_w1, xyz_b1, xyz_g1, xyz_be1, xyz_w2, xyz_b2, mlp_w1, mlp_b1, mlp_g1, mlp_be1, mlp_w2, mlp_b2):
    raise NotImplementedError("write your pallas kernel here")


# speedup vs baseline: 5.1439x; 5.1439x over previous
import jax
import jax.numpy as jnp
from jax.experimental import pallas as pl

_K = 6
_HF = 32
_HX = 16
_HM = 32
_H2 = 16
_GW = _HF + _HX


def _prep_body(sp_fea_ref, sp_xyz_ref, fw1_ref, fb1_ref, xw1_ref, xb1_ref,
               gcat_ref):
    gf = jnp.dot(sp_fea_ref[...], fw1_ref[...].T,
                 preferred_element_type=jnp.float32) + fb1_ref[...]
    gx = jnp.dot(sp_xyz_ref[...], xw1_ref[...].T,
                 preferred_element_type=jnp.float32) + xb1_ref[...]
    gcat_ref[...] = jnp.concatenate([gf, gx], axis=1)


def _ab_body(x_ref, xyz_ref, idx_ref, gcat_ref, fw1_ref, mw1_ref, mb1_ref,
             xw1_ref,
             bf_ref, bx_ref, bm_ref, hg_ref, stats_ref, stats2_ref):
    nb = x_ref.shape[0]
    m = gcat_ref.shape[0]
    x = x_ref[...]
    bf = jnp.dot(x, fw1_ref[...].T, preferred_element_type=jnp.float32)
    bm = jnp.dot(x, mw1_ref[...].T,
                 preferred_element_type=jnp.float32) + mb1_ref[...]
    bx = jnp.dot(xyz_ref[...], xw1_ref[...].T,
                 preferred_element_type=jnp.float32)
    bf_ref[...] = bf
    bm_ref[...] = bm
    bx_ref[...] = bx

    gcat = gcat_ref[...]
    lane = jax.lax.broadcasted_iota(jnp.int32, (nb, m), 1)
    sg = jnp.zeros((nb, _GW), jnp.float32)
    cnt = jnp.zeros((1, m), jnp.float32)
    for k in range(_K):
        pk = (idx_ref[:, k:k + 1] == lane).astype(jnp.float32)
        hk = jnp.dot(pk, gcat, preferred_element_type=jnp.float32)
        hg_ref[:, k * _GW:(k + 1) * _GW] = hk
        sg = sg + hk
        cnt = cnt + jnp.sum(pk, axis=0, keepdims=True)

    bcat = jnp.concatenate([bf, bx], axis=1)
    cross = jnp.sum(bcat * sg, axis=0, keepdims=True)

    @pl.when(pl.program_id(0) == 0)
    def _():
        stats_ref[...] = jnp.zeros_like(stats_ref)
        stats2_ref[...] = jnp.zeros_like(stats2_ref)

    stats_ref[0:1, 0:_HM] = stats_ref[0:1, 0:_HM] + jnp.sum(bm, 0, keepdims=True)
    stats_ref[1:2, 0:_HM] = stats_ref[1:2, 0:_HM] + jnp.sum(bm * bm, 0, keepdims=True)
    stats_ref[2:3, 0:_HF] = stats_ref[2:3, 0:_HF] + jnp.sum(bf, 0, keepdims=True)
    stats_ref[3:4, 0:_HF] = stats_ref[3:4, 0:_HF] + jnp.sum(bf * bf, 0, keepdims=True)
    stats_ref[4:5, 0:_HX] = stats_ref[4:5, 0:_HX] + jnp.sum(bx, 0, keepdims=True)
    stats_ref[5:6, 0:_HX] = stats_ref[5:6, 0:_HX] + jnp.sum(bx * bx, 0, keepdims=True)
    stats2_ref[0:1, :] = stats2_ref[0:1, :] + cnt
    stats2_ref[1:2, 0:_GW] = stats2_ref[1:2, 0:_GW] + cross


def _c_body(n_pts, stats_ref, stats2_ref, gcat_ref, fg1_ref, fbe1_ref,
            xg1_ref, xbe1_ref, mg1_ref, mbe1_ref, coefs_ref):
    nk = float(n_pts * _K)
    nf = float(n_pts)
    cnt = stats2_ref[0:1, :]
    gcat = gcat_ref[...]
    cnt_g = jnp.dot(cnt, gcat, preferred_element_type=jnp.float32)
    cnt_g2 = jnp.dot(cnt, gcat * gcat, preferred_element_type=jnp.float32)
    cross = stats2_ref[1:2, 0:_GW]

    coefs_ref[...] = jnp.zeros_like(coefs_ref)

    sbf = stats_ref[2:3, 0:_HF]
    sbf2 = stats_ref[3:4, 0:_HF]
    mean_f = cnt_g[:, 0:_HF] / nk - sbf / nf
    eh2_f = cnt_g2[:, 0:_HF] / nk - 2.0 * cross[:, 0:_HF] / nk + sbf2 / nf
    var_f = eh2_f - mean_f * mean_f
    a_f = fg1_ref[...] * jax.lax.rsqrt(var_f + 1e-5)
    coefs_ref[0:1, 0:_HF] = a_f
    coefs_ref[1:2, 0:_HF] = fbe1_ref[...] - a_f * mean_f

    sbx = stats_ref[4:5, 0:_HX]
    sbx2 = stats_ref[5:6, 0:_HX]
    mean_x = cnt_g[:, _HF:_GW] / nk - sbx / nf
    eh2_x = cnt_g2[:, _HF:_GW] / nk - 2.0 * cross[:, _HF:_GW] / nk + sbx2 / nf
    var_x = eh2_x - mean_x * mean_x
    a_x = xg1_ref[...] * jax.lax.rsqrt(var_x + 1e-5)
    coefs_ref[2:3, 0:_HX] = a_x
    coefs_ref[3:4, 0:_HX] = xbe1_ref[...] - a_x * mean_x

    sbm = stats_ref[0:1, 0:_HM]
    sbm2 = stats_ref[1:2, 0:_HM]
    mean_m = sbm / nf
    var_m = sbm2 / nf - mean_m * mean_m
    a_m = mg1_ref[...] * jax.lax.rsqrt(var_m + 1e-5)
    coefs_ref[4:5, 0:_HM] = a_m
    coefs_ref[5:6, 0:_HM] = mbe1_ref[...] - a_m * mean_m


def _l2n(v):
    nrm = jnp.sqrt(jnp.sum(v * v, axis=1, keepdims=True))
    return v / jnp.maximum(nrm, 1e-12)


def _d_body(hg_ref, bf_ref, bx_ref, bm_ref, idx_ref, x_ref, xyz_ref,
            coefs_ref, fw2_ref, fb2_ref, xw2_ref, xb2_ref, mw2_ref, mb2_ref,
            out_fea_ref, out_aux_ref):
    nb = x_ref.shape[0]
    m = out_fea_ref.shape[0]
    coefs = coefs_ref[...]
    a_f = coefs[0:1, 0:_HF]
    c_f = coefs[1:2, 0:_HF]
    a_x = coefs[2:3, 0:_HX]
    c_x = coefs[3:4, 0:_HX]
    a_m = coefs[4:5, 0:_HM]
    c_m = coefs[5:6, 0:_HM]

    hm = jnp.maximum(a_m * bm_ref[...] + c_m, 0.0)
    pfe = jnp.dot(hm, mw2_ref[...].T,
                  preferred_element_type=jnp.float32) + mb2_ref[...]
    pfe = _l2n(pfe)

    bf = bf_ref[...]
    bx = bx_ref[...]
    logits = []
    for k in range(_K):
        hk = hg_ref[:, k * _GW:(k + 1) * _GW]
        hf = jnp.maximum(a_f * (hk[:, 0:_HF] - bf) + c_f, 0.0)
        wf = jnp.dot(hf, fw2_ref[...].T,
                     preferred_element_type=jnp.float32) + fb2_ref[...]
        wf = _l2n(wf)
        hx = jnp.maximum(a_x * (hk[:, _HF:_GW] - bx) + c_x, 0.0)
        wx = jnp.dot(hx, xw2_ref[...].T,
                     preferred_element_type=jnp.float32) + xb2_ref[...]
        wx = _l2n(wx)
        nwf = jnp.sum(pfe * wf, axis=1, keepdims=True)
        nwx = jnp.sum(pfe * wx, axis=1, keepdims=True)
        logits.append(nwf * nwx)

    mx = logits[0]
    for k in range(1, _K):
        mx = jnp.maximum(mx, logits[k])
    exps = [jnp.exp(l - mx) for l in logits]
    ssum = exps[0]
    for k in range(1, _K):
        ssum = ssum + exps[k]

    lane = jax.lax.broadcasted_iota(jnp.int32, (nb, m), 1)
    s = jnp.zeros((nb, m), jnp.float32)
    for k in range(_K):
        biw = exps[k] / ssum
        s = s + biw * (idx_ref[:, k:k + 1] == lane).astype(jnp.float32)

    x8 = jnp.concatenate(
        [xyz_ref[...], jnp.ones((nb, 1), jnp.float32),
         jnp.zeros((nb, 4), jnp.float32)], axis=1)

    dn = (((0,), (0,)), ((), ()))
    upd_fea = jax.lax.dot_general(s, x_ref[...], dn,
                                  preferred_element_type=jnp.float32)
    upd_aux = jax.lax.dot_general(s, x8, dn,
                                  preferred_element_type=jnp.float32)

    @pl.when(pl.program_id(0) == 0)
    def _():
        out_fea_ref[...] = jnp.zeros_like(out_fea_ref)
        out_aux_ref[...] = jnp.zeros_like(out_aux_ref)

    out_fea_ref[...] = out_fea_ref[...] + upd_fea
    out_aux_ref[...] = out_aux_ref[...] + upd_aux

    @pl.when(pl.program_id(0) == pl.num_programs(0) - 1)
    def _():
        den = out_aux_ref[:, 3:4] + 1e-8
        out_fea_ref[...] = out_fea_ref[...] / den
        out_aux_ref[...] = out_aux_ref[...] / den


def kernel(sp_fea, sp_xyz, o_p_fea, p_xyz, c2p_idx_abs, c2p_idx, cluster_idx,
           offset, sp_offset,
           fea_w1, fea_b1, fea_g1, fea_be1, fea_w2, fea_b2,
           xyz_w1, xyz_b1, xyz_g1, xyz_be1, xyz_w2, xyz_b2,
           mlp_w1, mlp_b1, mlp_g1, mlp_be1, mlp_w2, mlp_b2):
    n, c = o_p_fea.shape
    m = sp_fea.shape[0]
    f32 = jnp.float32

    nb = n
    for cand in (1000, 500, 250, 200, 100, 50, 25, 10, 8, 5, 4, 2, 1):
        if n % cand == 0:
            nb = cand
            break
    grid = n // nb

    r = lambda v: v.reshape(1, -1)

    gcat = pl.pallas_call(
        _prep_body,
        out_shape=jax.ShapeDtypeStruct((m, _GW), f32),
    )(sp_fea, sp_xyz, fea_w1, r(fea_b1), xyz_w1, r(xyz_b1))

    blk = lambda shape: pl.BlockSpec(shape, lambda i: (i, 0))
    full = lambda shape: pl.BlockSpec(shape, lambda i: (0, 0))

    bf, bx, bm, hg, stats, stats2 = pl.pallas_call(
        _ab_body,
        grid=(grid,),
        in_specs=[
            blk((nb, c)), blk((nb, 3)), blk((nb, _K)), full((m, _GW)),
            full((_HF, c)), full((_HM, c)), full((1, _HM)), full((_HX, 3)),
        ],
        out_specs=[
            blk((nb, _HF)), blk((nb, _HX)), blk((nb, _HM)),
            blk((nb, _K * _GW)), full((8, 128)), full((8, m)),
        ],
        out_shape=[
            jax.ShapeDtypeStruct((n, _HF), f32),
            jax.ShapeDtypeStruct((n, _HX), f32),
            jax.ShapeDtypeStruct((n, _HM), f32),
            jax.ShapeDtypeStruct((n, _K * _GW), f32),
            jax.ShapeDtypeStruct((8, 128), f32),
            jax.ShapeDtypeStruct((8, m), f32),
        ],
    )(o_p_fea, p_xyz, c2p_idx_abs, gcat,
      fea_w1, mlp_w1, r(mlp_b1), xyz_w1)

    import functools
    coefs = pl.pallas_call(
        functools.partial(_c_body, n),
        out_shape=jax.ShapeDtypeStruct((8, 128), f32),
    )(stats, stats2, gcat, r(fea_g1), r(fea_be1), r(xyz_g1), r(xyz_be1),
      r(mlp_g1), r(mlp_be1))

    out_fea, out_aux = pl.pallas_call(
        _d_body,
        grid=(grid,),
        in_specs=[
            blk((nb, _K * _GW)), blk((nb, _HF)), blk((nb, _HX)),
            blk((nb, _HM)), blk((nb, _K)), blk((nb, c)), blk((nb, 3)),
            full((8, 128)),
            full((_H2, _HF)), full((1, _H2)),
            full((_H2, _HX)), full((1, _H2)),
            full((_H2, _HM)), full((1, _H2)),
        ],
        out_specs=[full((m, c)), full((m, 8))],
        out_shape=[
            jax.ShapeDtypeStruct((m, c), f32),
            jax.ShapeDtypeStruct((m, 8), f32),
        ],
    )(hg, bf, bx, bm, c2p_idx, o_p_fea, p_xyz, coefs,
      fea_w2, r(fea_b2), xyz_w2, r(xyz_b2), mlp_w2, r(mlp_b2))

    return (out_fea, out_aux[:, :3])


# speedup vs baseline: 8.0828x; 1.5713x over previous
import functools

import jax
import jax.numpy as jnp
from jax.experimental import pallas as pl

_K = 6
_HF = 32
_HX = 16
_HM = 32
_H2 = 16
_GW = _HF + _HX
_GWK = _GW * _K
_H2K = _H2 * _K


def _prep_body(sp_fea_ref, sp_xyz_ref, fw1_ref, fb1_ref, xw1_ref, xb1_ref,
               gcat_ref):
    gf = jnp.dot(sp_fea_ref[...], fw1_ref[...].T,
                 preferred_element_type=jnp.float32) + fb1_ref[...]
    gx = jnp.dot(sp_xyz_ref[...], xw1_ref[...].T,
                 preferred_element_type=jnp.float32) + xb1_ref[...]
    gcat_ref[...] = jnp.concatenate([gf, gx], axis=1)


def _ab_body(x_ref, xyz_ref, idx_ref, gcat_ref, fw1_ref, mw1_ref, mb1_ref,
             xw1_ref,
             bm_ref, hg_ref, stats_ref):
    nb = x_ref.shape[0]
    m = gcat_ref.shape[0]
    x = x_ref[...]
    bf = jnp.dot(x, fw1_ref[...].T, preferred_element_type=jnp.float32)
    bm = jnp.dot(x, mw1_ref[...].T,
                 preferred_element_type=jnp.float32) + mb1_ref[...]
    bx = jnp.dot(xyz_ref[...], xw1_ref[...].T,
                 preferred_element_type=jnp.float32)
    bm_ref[...] = bm
    bcat = jnp.concatenate([bf, bx], axis=1)

    gcat = gcat_ref[...]
    lane = jax.lax.broadcasted_iota(jnp.int32, (nb, m), 1)
    sh = jnp.zeros((1, _GW), jnp.float32)
    sh2 = jnp.zeros((1, _GW), jnp.float32)
    for k in range(_K):
        pk = (idx_ref[:, k:k + 1] == lane).astype(jnp.float32)
        hk = jnp.dot(pk, gcat, preferred_element_type=jnp.float32) - bcat
        hg_ref[:, k * _GW:(k + 1) * _GW] = hk
        sh = sh + jnp.sum(hk, axis=0, keepdims=True)
        sh2 = sh2 + jnp.sum(hk * hk, axis=0, keepdims=True)

    @pl.when(pl.program_id(0) == 0)
    def _():
        stats_ref[...] = jnp.zeros_like(stats_ref)

    stats_ref[0:1, 0:_GW] = stats_ref[0:1, 0:_GW] + sh
    stats_ref[1:2, 0:_GW] = stats_ref[1:2, 0:_GW] + sh2
    stats_ref[2:3, 0:_HM] = stats_ref[2:3, 0:_HM] + jnp.sum(bm, 0, keepdims=True)
    stats_ref[3:4, 0:_HM] = stats_ref[3:4, 0:_HM] + jnp.sum(bm * bm, 0, keepdims=True)


def _c_body(n_pts, stats_ref, fg1_ref, fbe1_ref, xg1_ref, xbe1_ref,
            mg1_ref, mbe1_ref, coefs_ref):
    nk = float(n_pts * _K)
    nf = float(n_pts)
    coefs_ref[...] = jnp.zeros_like(coefs_ref)

    g48 = jnp.concatenate([fg1_ref[...], xg1_ref[...]], axis=1)
    be48 = jnp.concatenate([fbe1_ref[...], xbe1_ref[...]], axis=1)
    mean_h = stats_ref[0:1, 0:_GW] / nk
    var_h = stats_ref[1:2, 0:_GW] / nk - mean_h * mean_h
    a48 = g48 * jax.lax.rsqrt(var_h + 1e-5)
    c48 = be48 - a48 * mean_h
    for k in range(_K):
        coefs_ref[0:1, k * _GW:(k + 1) * _GW] = a48
        coefs_ref[1:2, k * _GW:(k + 1) * _GW] = c48

    mean_m = stats_ref[2:3, 0:_HM] / nf
    var_m = stats_ref[3:4, 0:_HM] / nf - mean_m * mean_m
    a_m = mg1_ref[...] * jax.lax.rsqrt(var_m + 1e-5)
    coefs_ref[2:3, 0:_HM] = a_m
    coefs_ref[3:4, 0:_HM] = mbe1_ref[...] - a_m * mean_m


def _d_body(hg_ref, bm_ref, idx_ref, x_ref, xyz_ref, coefs_ref,
            wbf_ref, wbx_ref, itile_ref, fb2t_ref, xb2t_ref,
            mw2_ref, mb2_ref, sel_ref,
            out_ref):
    nb = x_ref.shape[0]
    m = out_ref.shape[0]
    a288 = coefs_ref[0:1, :]
    c288 = coefs_ref[1:2, :]
    a_m = coefs_ref[2:3, 0:_HM]
    c_m = coefs_ref[3:4, 0:_HM]

    hm = jnp.maximum(a_m * bm_ref[...] + c_m, 0.0)
    p = jnp.dot(hm, mw2_ref[...].T,
                preferred_element_type=jnp.float32) + mb2_ref[...]
    pp = jnp.sum(p * p, axis=1, keepdims=True)
    pn = jnp.maximum(jnp.sqrt(pp), 1e-12)
    prep6 = jnp.dot(p, itile_ref[...], preferred_element_type=jnp.float32)

    ht = jnp.maximum(hg_ref[...] * a288 + c288, 0.0)
    wf = jnp.dot(ht, wbf_ref[...],
                 preferred_element_type=jnp.float32) + fb2t_ref[...]
    wx = jnp.dot(ht, wbx_ref[...],
                 preferred_element_type=jnp.float32) + xb2t_ref[...]

    sel = sel_ref[...]
    u = jnp.dot(wf * prep6, sel, preferred_element_type=jnp.float32)
    v = jnp.dot(wx * prep6, sel, preferred_element_type=jnp.float32)
    s2 = jnp.dot(wf * wf, sel, preferred_element_type=jnp.float32)
    t2 = jnp.dot(wx * wx, sel, preferred_element_type=jnp.float32)

    df = jnp.maximum(jnp.sqrt(s2[:, 0:_K]), 1e-12)
    dx = jnp.maximum(jnp.sqrt(t2[:, 0:_K]), 1e-12)
    logits = (u[:, 0:_K] * v[:, 0:_K]) / (df * dx * (pn * pn))

    mx = jnp.max(logits, axis=1, keepdims=True)
    e = jnp.exp(logits - mx)
    biw = e / jnp.sum(e, axis=1, keepdims=True)

    lane = jax.lax.broadcasted_iota(jnp.int32, (nb, m), 1)
    s = jnp.zeros((nb, m), jnp.float32)
    for k in range(_K):
        s = s + biw[:, k:k + 1] * (idx_ref[:, k:k + 1] == lane).astype(jnp.float32)

    x136 = jnp.concatenate(
        [x_ref[...], xyz_ref[...], jnp.ones((nb, 1), jnp.float32),
         jnp.zeros((nb, 4), jnp.float32)], axis=1)

    dn = (((0,), (0,)), ((), ()))
    upd = jax.lax.dot_general(s, x136, dn, preferred_element_type=jnp.float32)

    @pl.when(pl.program_id(0) == 0)
    def _():
        out_ref[...] = jnp.zeros_like(out_ref)

    out_ref[...] = out_ref[...] + upd

    @pl.when(pl.program_id(0) == pl.num_programs(0) - 1)
    def _():
        den = out_ref[:, 131:132] + 1e-8
        out_ref[...] = out_ref[...] / den


def kernel(sp_fea, sp_xyz, o_p_fea, p_xyz, c2p_idx_abs, c2p_idx, cluster_idx,
           offset, sp_offset,
           fea_w1, fea_b1, fea_g1, fea_be1, fea_w2, fea_b2,
           xyz_w1, xyz_b1, xyz_g1, xyz_be1, xyz_w2, xyz_b2,
           mlp_w1, mlp_b1, mlp_g1, mlp_be1, mlp_w2, mlp_b2):
    n, c = o_p_fea.shape
    m = sp_fea.shape[0]
    f32 = jnp.float32

    nb = n
    for cand in (1000, 500, 250, 200, 100, 50, 25, 10, 8, 5, 4, 2, 1):
        if n % cand == 0:
            nb = cand
            break
    grid = n // nb

    r = lambda v: v.reshape(1, -1)

    zf = jnp.zeros((_GWK, _H2K), f32)
    zx = jnp.zeros((_GWK, _H2K), f32)
    it = jnp.zeros((_H2, _H2K), f32)
    sel = jnp.zeros((_H2K, 8), f32)
    eye16 = jnp.eye(_H2, dtype=f32)
    for k in range(_K):
        zf = zf.at[k * _GW:k * _GW + _HF, k * _H2:(k + 1) * _H2].set(fea_w2.T)
        zx = zx.at[k * _GW + _HF:(k + 1) * _GW, k * _H2:(k + 1) * _H2].set(xyz_w2.T)
        it = it.at[:, k * _H2:(k + 1) * _H2].set(eye16)
        sel = sel.at[k * _H2:(k + 1) * _H2, k].set(1.0)
    fb2t = jnp.tile(fea_b2, _K).reshape(1, _H2K)
    xb2t = jnp.tile(xyz_b2, _K).reshape(1, _H2K)

    gcat = pl.pallas_call(
        _prep_body,
        out_shape=jax.ShapeDtypeStruct((m, _GW), f32),
    )(sp_fea, sp_xyz, fea_w1, r(fea_b1), xyz_w1, r(xyz_b1))

    blk = lambda shape: pl.BlockSpec(shape, lambda i: (i, 0))
    full = lambda shape: pl.BlockSpec(shape, lambda i: (0, 0))

    bm, hg, stats = pl.pallas_call(
        _ab_body,
        grid=(grid,),
        in_specs=[
            blk((nb, c)), blk((nb, 3)), blk((nb, _K)), full((m, _GW)),
            full((_HF, c)), full((_HM, c)), full((1, _HM)), full((_HX, 3)),
        ],
        out_specs=[
            blk((nb, _HM)), blk((nb, _GWK)), full((8, 128)),
        ],
        out_shape=[
            jax.ShapeDtypeStruct((n, _HM), f32),
            jax.ShapeDtypeStruct((n, _GWK), f32),
            jax.ShapeDtypeStruct((8, 128), f32),
        ],
    )(o_p_fea, p_xyz, c2p_idx_abs, gcat,
      fea_w1, mlp_w1, r(mlp_b1), xyz_w1)

    coefs = pl.pallas_call(
        functools.partial(_c_body, n),
        out_shape=jax.ShapeDtypeStruct((8, _GWK), f32),
    )(stats, r(fea_g1), r(fea_be1), r(xyz_g1), r(xyz_be1),
      r(mlp_g1), r(mlp_be1))

    out = pl.pallas_call(
        _d_body,
        grid=(grid,),
        in_specs=[
            blk((nb, _GWK)), blk((nb, _HM)), blk((nb, _K)),
            blk((nb, c)), blk((nb, 3)),
            full((8, _GWK)),
            full((_GWK, _H2K)), full((_GWK, _H2K)), full((_H2, _H2K)),
            full((1, _H2K)), full((1, _H2K)),
            full((_H2, _HM)), full((1, _H2)), full((_H2K, 8)),
        ],
        out_specs=[full((m, 136))],
        out_shape=[jax.ShapeDtypeStruct((m, 136), f32)],
    )(hg, bm, c2p_idx, o_p_fea, p_xyz, coefs,
      zf, zx, it, fb2t, xb2t, mlp_w2, r(mlp_b2), sel)

    out = out[0]
    return (out[:, :c], out[:, c:c + 3])
